# Initial kernel scaffold; baseline (speedup 1.0000x reference)
import jax
import jax.numpy as jnp
from jax.experimental import pallas as pl


def kernel(edge_hidden, q_rel_emb, line_src, line_dst, n_edge, W_src, W_dst, W_qr, b_qr, w_alpha, b_alpha, W_msg, W_out):
    raise NotImplementedError("write your pallas kernel here")


# speedup vs baseline: 6.8306x; 6.8306x over previous
import jax
import jax.numpy as jnp
from jax import lax
from jax.experimental import pallas as pl
from jax.experimental.pallas import tpu as pltpu
from jax.experimental.pallas import tpu_sc as plsc

N_EDGE = 10000
N_LINE = 320000
HIDDEN = 128
ATTN = 64

NC = 2
NS = 16
NW = NC * NS
LINES_PER_TILE = N_LINE // NW
CHUNK = 80
NCHUNK = LINES_PER_TILE // CHUNK
N_PAD = 10240
ROWS_PER_TILE = N_PAD // NS
ZROWS = 128


def _proj_body(eh_ref, qr_ref, ws_ref, wd_ref, wq_ref, bq_ref, wm_ref,
               p_ref, m_ref):
    eh = eh_ref[...]
    qr = qr_ref[...]
    a = jnp.dot(eh, ws_ref[...], preferred_element_type=jnp.float32)
    bc = (jnp.dot(eh, wd_ref[...], preferred_element_type=jnp.float32)
          + jnp.dot(qr, wq_ref[...], preferred_element_type=jnp.float32)
          + bq_ref[...])
    p_ref[...] = jnp.concatenate([a, bc], axis=1)
    m_ref[...] = jnp.dot(eh, wm_ref[...], preferred_element_type=jnp.float32)


def _out_body(p0_ref, p1_ref, wo_ref, o_ref):
    o_ref[...] = jnp.dot(p0_ref[...] + p1_ref[...], wo_ref[...],
                         preferred_element_type=jnp.float32)


def _sc_body(p_hbm, m_hbm, src_hbm, dst_hbm, wa_hbm, ba_hbm,
             out_hbm, agg_sh, idx_s, idx_d, a_v, bc_v, m_v, wa_v, ba_v,
             zb_v, sem):
    cid = lax.axis_index("c")
    sid = lax.axis_index("s")
    wid = cid * NS + sid

    def zrow(r, carry):
        for k in range(HIDDEN // 16):
            zb_v[r, pl.ds(k * 16, 16)] = jnp.zeros((16,), jnp.float32)
        return carry
    lax.fori_loop(0, ZROWS, zrow, 0)
    for j in range(ROWS_PER_TILE // ZROWS):
        pltpu.sync_copy(zb_v, agg_sh.at[pl.ds(sid * ROWS_PER_TILE + j * ZROWS, ZROWS)])

    pltpu.sync_copy(wa_hbm, wa_v)
    pltpu.sync_copy(ba_hbm, ba_v)
    plsc.subcore_barrier()

    def chunk_body(t, carry):
        base = pl.multiple_of(wid * LINES_PER_TILE + t * CHUNK, 16)
        pltpu.sync_copy(src_hbm.at[pl.ds(base, CHUNK)], idx_s)
        pltpu.sync_copy(dst_hbm.at[pl.ds(base, CHUNK)], idx_d)
        ca = pltpu.async_copy(p_hbm.at[idx_s], a_v, sem)
        cb = pltpu.async_copy(p_hbm.at[idx_d], bc_v, sem)
        cm = pltpu.async_copy(m_hbm.at[idx_s], m_v, sem)
        ca.wait()
        cb.wait()
        cm.wait()
        wa = [wa_v[pl.ds(k * 16, 16)] for k in range(ATTN // 16)]
        bvec = ba_v[...]

        def line_body(i, c2):
            p = []
            for k in range(ATTN // 16):
                pre = a_v[i, pl.ds(k * 16, 16)] + bc_v[i, pl.ds(ATTN + k * 16, 16)]
                p.append(jnp.maximum(pre, 0.0) * wa[k])
            s = jnp.sum((p[0] + p[1]) + (p[2] + p[3]))
            z = s + bvec
            alpha = 1.0 / (1.0 + jnp.exp(-z))
            for k in range(HIDDEN // 16):
                m_v[i, pl.ds(k * 16, 16)] = m_v[i, pl.ds(k * 16, 16)] * alpha
            return c2
        lax.fori_loop(0, CHUNK, line_body, 0)
        pltpu.sync_copy(m_v, agg_sh.at[idx_d], add=True)
        return carry
    lax.fori_loop(0, NCHUNK, chunk_body, 0)

    plsc.subcore_barrier()
    pltpu.sync_copy(agg_sh.at[pl.ds(sid * ROWS_PER_TILE, ROWS_PER_TILE)],
                    out_hbm.at[cid, pl.ds(sid * ROWS_PER_TILE, ROWS_PER_TILE)])


def kernel(edge_hidden, q_rel_emb, line_src, line_dst, n_edge, W_src, W_dst,
           W_qr, b_qr, w_alpha, b_alpha, W_msg, W_out):
    n = edge_hidden.shape[0]
    blk = 1000
    grid = n // blk

    p_proj, m_proj = pl.pallas_call(
        _proj_body,
        grid=(grid,),
        in_specs=[
            pl.BlockSpec((blk, HIDDEN), lambda i: (i, 0)),
            pl.BlockSpec((blk, HIDDEN), lambda i: (i, 0)),
            pl.BlockSpec((HIDDEN, ATTN), lambda i: (0, 0)),
            pl.BlockSpec((HIDDEN, ATTN), lambda i: (0, 0)),
            pl.BlockSpec((HIDDEN, ATTN), lambda i: (0, 0)),
            pl.BlockSpec((1, ATTN), lambda i: (0, 0)),
            pl.BlockSpec((HIDDEN, HIDDEN), lambda i: (0, 0)),
        ],
        out_specs=[
            pl.BlockSpec((blk, 2 * ATTN), lambda i: (i, 0)),
            pl.BlockSpec((blk, HIDDEN), lambda i: (i, 0)),
        ],
        out_shape=[
            jax.ShapeDtypeStruct((n, 2 * ATTN), jnp.float32),
            jax.ShapeDtypeStruct((n, HIDDEN), jnp.float32),
        ],
    )(edge_hidden, q_rel_emb, W_src, W_dst, W_qr, b_qr.reshape(1, ATTN), W_msg)

    wa_flat = w_alpha.reshape(ATTN)
    ba_vec = jnp.broadcast_to(b_alpha.reshape(1), (16,))
    src32 = line_src.astype(jnp.int32)
    dst32 = line_dst.astype(jnp.int32)

    parts = pl.kernel(
        _sc_body,
        out_type=jax.ShapeDtypeStruct((NC, N_PAD, HIDDEN), jnp.float32),
        mesh=plsc.VectorSubcoreMesh(core_axis_name="c", subcore_axis_name="s",
                                    num_cores=NC, num_subcores=NS),
        compiler_params=pltpu.CompilerParams(needs_layout_passes=False),
        scratch_types=[
            pltpu.VMEM_SHARED((N_PAD, HIDDEN), jnp.float32),
            pltpu.VMEM((CHUNK,), jnp.int32),
            pltpu.VMEM((CHUNK,), jnp.int32),
            pltpu.VMEM((CHUNK, 2 * ATTN), jnp.float32),
            pltpu.VMEM((CHUNK, 2 * ATTN), jnp.float32),
            pltpu.VMEM((CHUNK, HIDDEN), jnp.float32),
            pltpu.VMEM((ATTN,), jnp.float32),
            pltpu.VMEM((16,), jnp.float32),
            pltpu.VMEM((ZROWS, HIDDEN), jnp.float32),
            pltpu.SemaphoreType.DMA,
        ],
    )(p_proj, m_proj, src32, dst32, wa_flat, ba_vec)

    hidden_new = pl.pallas_call(
        _out_body,
        grid=(grid,),
        in_specs=[
            pl.BlockSpec((blk, HIDDEN), lambda i: (i, 0)),
            pl.BlockSpec((blk, HIDDEN), lambda i: (i, 0)),
            pl.BlockSpec((HIDDEN, HIDDEN), lambda i: (0, 0)),
        ],
        out_specs=pl.BlockSpec((blk, HIDDEN), lambda i: (i, 0)),
        out_shape=jax.ShapeDtypeStruct((n, HIDDEN), jnp.float32),
    )(parts[0], parts[1], W_out)

    return hidden_new + jnp.zeros((), dtype=hidden_new.dtype) * n_edge
